# Initial kernel scaffold; baseline (speedup 1.0000x reference)
import jax
import jax.numpy as jnp
from jax.experimental import pallas as pl


def kernel(x, embed_table, W_lin, bow_bias):
    raise NotImplementedError("write your pallas kernel here")


# speedup vs baseline: 5.4593x; 5.4593x over previous
import functools

import jax
import jax.numpy as jnp
from jax import lax
from jax.experimental import pallas as pl
from jax.experimental.pallas import tpu as pltpu
from jax.experimental.pallas import tpu_sc as plsc

BATCH = 4096
HIST = 50
EMBED = 64
TAGS = 1000

NC, NS, LANES = 2, 16, 16
NW = NC * NS
B_PER_W = BATCH // NW
CHUNK_B = 2
CHUNK_I = CHUNK_B * HIST
N_CHUNKS = B_PER_W // CHUNK_B
NBUF = 4
EV = EMBED // LANES


def _sc_pool_body(x_hbm, table_hbm, out_hbm, idx_v, rows_v, out_v, sems):
    wid = lax.axis_index("s") * NC + lax.axis_index("c")
    pltpu.sync_copy(x_hbm.at[wid], idx_v)

    def start(g, b):
        pltpu.async_copy(table_hbm.at[idx_v.at[g]], rows_v.at[b], sems.at[b])

    for b in range(NBUF):
        start(b, b)

    def outer(t, carry):
        for b in range(NBUF):
            g = t * NBUF + b
            pltpu.make_async_copy(
                table_hbm.at[idx_v.at[g]], rows_v.at[b], sems.at[b]
            ).wait()
            for r in range(CHUNK_B):
                row = g * CHUNK_B + r
                for k in range(EV):
                    acc = rows_v[b, r * HIST, pl.ds(k * LANES, LANES)]
                    for j in range(1, HIST):
                        acc = acc + rows_v[b, r * HIST + j, pl.ds(k * LANES, LANES)]
                    out_v[row, pl.ds(k * LANES, LANES)] = acc
            nxt = g + NBUF

            @pl.when(nxt < N_CHUNKS)
            def _():
                start(nxt, b)

        return carry

    lax.fori_loop(0, N_CHUNKS // NBUF, outer, 0)
    pltpu.sync_copy(out_v, out_hbm.at[wid])


@functools.cache
def _sc_pool():
    return functools.partial(
        pl.kernel,
        out_type=jax.ShapeDtypeStruct((NW, B_PER_W, EMBED), jnp.float32),
        mesh=plsc.VectorSubcoreMesh(core_axis_name="c", subcore_axis_name="s"),
        compiler_params=pltpu.CompilerParams(use_tc_tiling_on_sc=False),
        scratch_types=[
            pltpu.VMEM((N_CHUNKS, CHUNK_I), jnp.int32),
            pltpu.VMEM((NBUF, CHUNK_I, EMBED), jnp.float32),
            pltpu.VMEM((B_PER_W, EMBED), jnp.float32),
            pltpu.SemaphoreType.DMA((NBUF,)),
        ],
    )(_sc_pool_body)


BM = 512


def _dense_body(p_ref, w_ref, b_ref, o_ref):
    x = p_ref[...]
    w = w_ref[...]
    s = lax.dot_general(
        x, w, (((1,), (1,)), ((), ())), preferred_element_type=jnp.float32
    )
    s = s + b_ref[...]
    m = jnp.max(s, axis=-1, keepdims=True)
    e = jnp.exp(s - m)
    lse = jnp.log(jnp.sum(e, axis=-1, keepdims=True)) + m
    o_ref[...] = s - lse


_dense = pl.pallas_call(
    _dense_body,
    grid=(BATCH // BM,),
    in_specs=[
        pl.BlockSpec((BM, EMBED), lambda i: (i, 0)),
        pl.BlockSpec((TAGS, EMBED), lambda i: (0, 0)),
        pl.BlockSpec((1, TAGS), lambda i: (0, 0)),
    ],
    out_specs=pl.BlockSpec((BM, TAGS), lambda i: (i, 0)),
    out_shape=jax.ShapeDtypeStruct((BATCH, TAGS), jnp.float32),
    compiler_params=pltpu.CompilerParams(dimension_semantics=("parallel",)),
)


def kernel(x, embed_table, W_lin, bow_bias):
    x32 = x.astype(jnp.int32).reshape(NW, N_CHUNKS, CHUNK_I)
    pooled = _sc_pool()(x32, embed_table)
    pooled = pooled.reshape(BATCH, EMBED)
    return _dense(pooled, W_lin, bow_bias.reshape(1, TAGS))
